# Initial kernel scaffold; baseline (speedup 1.0000x reference)
import jax
import jax.numpy as jnp
from jax.experimental import pallas as pl


def kernel(x, edge_index, batch, edge_attr, W_enc, b_enc, W_convs, b_convs, gamma, beta, W_clf, b_clf):
    raise NotImplementedError("write your pallas kernel here")


# speedup vs baseline: 7.2222x; 7.2222x over previous
import functools

import jax
import jax.numpy as jnp
from jax import lax
from jax.experimental import pallas as pl
from jax.experimental.pallas import tpu as pltpu
from jax.experimental.pallas import tpu_sc as plsc

N = 10000
E = 160000
D = 256
DH = 128
NG = 16
NCORE = 2
NSUB = 16
CH = 128
R_SMALL = 624
R_LAST = N - (NSUB - 1) * R_SMALL

@functools.lru_cache(maxsize=None)
def _mesh():
    return plsc.VectorSubcoreMesh(
        core_axis_name="c", subcore_axis_name="s",
        num_cores=NCORE, num_subcores=NSUB)


def _sc_agg_body(u_hbm, src_hbm, dst_hbm, y_hbm,
                 srcv, dstv, rows, acc_sh, sem):
    c = lax.axis_index("c")
    s = lax.axis_index("s")
    r0 = s * R_SMALL

    def _init(nrows):
        pltpu.sync_copy(u_hbm.at[pl.ds(c * N + r0, nrows)],
                        acc_sh.at[pl.ds(r0, nrows)])

    @pl.when(s < NSUB - 1)
    def _():
        _init(R_SMALL)

    @pl.when(s == NSUB - 1)
    def _():
        _init(R_LAST)

    plsc.subcore_barrier()
    nchunk = E // CH

    def chunk_step(j, carry):
        chunk = j * NSUB + s

        @pl.when(chunk < nchunk)
        def _():
            off = pl.multiple_of(chunk * CH, CH)
            pltpu.sync_copy(src_hbm.at[pl.ds(c * E + off, CH)], srcv)
            pltpu.sync_copy(dst_hbm.at[pl.ds(off, CH)], dstv)
            pltpu.async_copy(u_hbm.at[srcv], rows, sem).wait()
            pltpu.sync_copy(rows, acc_sh.at[dstv], add=True)
        return carry

    lax.fori_loop(0, (nchunk + NSUB - 1) // NSUB, chunk_step, 0)
    plsc.subcore_barrier()

    def _wb(nrows):
        pltpu.sync_copy(acc_sh.at[pl.ds(r0, nrows)],
                        y_hbm.at[pl.ds(c * N + r0, nrows)])

    @pl.when(s < NSUB - 1)
    def _():
        _wb(R_SMALL)

    @pl.when(s == NSUB - 1)
    def _():
        _wb(R_LAST)


@functools.lru_cache(maxsize=None)
def _sc_agg():
    return pl.kernel(
        _sc_agg_body,
        out_type=jax.ShapeDtypeStruct((NCORE * N, DH), jnp.float32),
        mesh=_mesh(),
        scratch_types=[
            pltpu.VMEM((CH,), jnp.int32),
            pltpu.VMEM((CH,), jnp.int32),
            pltpu.VMEM((CH, DH), jnp.float32),
            pltpu.VMEM_SHARED((N, DH), jnp.float32),
            pltpu.SemaphoreType.DMA,
        ],
    )


def _split_halves(u_ref, u):
    u_ref[0:N, :] = u[:, 0:DH]
    u_ref[N:2 * N, :] = u[:, DH:D]


def _tc_enc_body(x_ref, w_ref, degp_ref, u_ref, dis_ref):
    deg = degp_ref[0:N, 0:1]
    dis = lax.rsqrt(deg)
    dis_ref[...] = dis
    h = jnp.dot(x_ref[...], w_ref[...], preferred_element_type=jnp.float32)
    _split_halves(u_ref, h * dis)


_tc_enc = pl.pallas_call(
    _tc_enc_body,
    out_shape=[jax.ShapeDtypeStruct((NCORE * N, DH), jnp.float32),
               jax.ShapeDtypeStruct((N, 1), jnp.float32)],
)


def _tc_mid_body(has_bn, y_ref, dis_ref, b_ref, w_ref, g_ref, be_ref, u_ref):
    dis = dis_ref[...]
    h = jnp.concatenate([y_ref[0:N, :], y_ref[N:2 * N, :]], axis=1)
    h = h * dis + b_ref[...]
    if has_bn:
        m = jnp.mean(h, axis=0, keepdims=True)
        v = jnp.mean((h - m) ** 2, axis=0, keepdims=True)
        h = (h - m) * lax.rsqrt(v + 1e-5) * g_ref[...] + be_ref[...]
        h = jnp.maximum(h, 0.0)
    h2 = jnp.dot(h, w_ref[...], preferred_element_type=jnp.float32)
    _split_halves(u_ref, h2 * dis)


_tc_mid_bn = pl.pallas_call(
    functools.partial(_tc_mid_body, True),
    out_shape=jax.ShapeDtypeStruct((NCORE * N, DH), jnp.float32),
)
_tc_mid_plain = pl.pallas_call(
    functools.partial(_tc_mid_body, False),
    out_shape=jax.ShapeDtypeStruct((NCORE * N, DH), jnp.float32),
)


def _tc_fin_body(y_ref, dis_ref, b_ref, batch_ref, wc_ref, bc_ref, out_ref):
    dis = dis_ref[...]
    h = jnp.concatenate([y_ref[0:N, :], y_ref[N:2 * N, :]], axis=1)
    h = h * dis + b_ref[...]
    onehot = (batch_ref[...] == lax.broadcasted_iota(jnp.int32, (N, NG), 1)
              ).astype(jnp.float32)
    ssum = lax.dot_general(onehot, h, (((0,), (0,)), ((), ())),
                           preferred_element_type=jnp.float32)
    cnt = jnp.sum(onehot, axis=0, keepdims=True)
    pooled = ssum / jnp.maximum(cnt.T, 1.0)
    out_ref[...] = jnp.dot(pooled, wc_ref[...],
                           preferred_element_type=jnp.float32) + bc_ref[...]


_tc_fin = pl.pallas_call(
    _tc_fin_body,
    out_shape=jax.ShapeDtypeStruct((NG, 64), jnp.float32),
)


def kernel(x, edge_index, batch, edge_attr, W_enc, b_enc, W_convs, b_convs,
           gamma, beta, W_clf, b_clf):
    src = edge_index[0].astype(jnp.int32)
    dst = edge_index[1].astype(jnp.int32)
    src2 = jnp.concatenate([src, src + N])

    sc_agg = _sc_agg()
    degp = sc_agg(jnp.ones((NCORE * N, DH), jnp.float32), src2, dst)
    u, dis = _tc_enc(x, W_enc, degp)

    y = sc_agg(u, src2, dst)
    u = _tc_mid_plain(y, dis, b_enc.reshape(1, D), W_convs[0],
                      gamma[0].reshape(1, D), beta[0].reshape(1, D))
    y = sc_agg(u, src2, dst)
    u = _tc_mid_bn(y, dis, b_convs[0].reshape(1, D), W_convs[2],
                   gamma[0].reshape(1, D), beta[0].reshape(1, D))
    y = sc_agg(u, src2, dst)
    u = _tc_mid_plain(y, dis, b_convs[2].reshape(1, D), W_convs[1],
                      gamma[0].reshape(1, D), beta[0].reshape(1, D))
    y = sc_agg(u, src2, dst)
    u = _tc_mid_bn(y, dis, b_convs[1].reshape(1, D), W_convs[2],
                   gamma[1].reshape(1, D), beta[1].reshape(1, D))
    y = sc_agg(u, src2, dst)
    out = _tc_fin(y, dis, b_convs[2].reshape(1, D),
                  batch.astype(jnp.int32).reshape(N, 1), W_clf,
                  b_clf.reshape(1, 64))
    return out
